# baseline (device time: 15067 ns/iter reference)
import jax
import jax.numpy as jnp
from jax import lax
from jax.experimental import pallas as pl
from jax.experimental.pallas import tpu as pltpu

N_DEV = 16
BLK = 64

_OFFSET_ORDER = sorted(range(1, N_DEV), key=lambda s: min(s, N_DEV - s))


def kernel(x, w_mat):
    m_glob, k_loc = x.shape
    k_glob, n = w_mat.shape
    assert k_loc == BLK and m_glob == N_DEV * BLK

    def body(x_ref, w_hbm_ref, out_ref, w_ref, xg_ref,
             w_copy_sem, send_sems, recv_sems):
        my = lax.axis_index("i")

        w_copy = pltpu.make_async_copy(w_hbm_ref, w_ref, w_copy_sem)
        w_copy.start()

        barrier = pltpu.get_barrier_semaphore()
        for s in range(1, N_DEV):
            peer = lax.rem(my + s, N_DEV)
            pl.semaphore_signal(
                barrier, inc=1,
                device_id=(peer,), device_id_type=pl.DeviceIdType.MESH,
            )
        pl.semaphore_wait(barrier, N_DEV - 1)

        rdmas = []
        for s in _OFFSET_ORDER:
            tgt = lax.rem(my + s, N_DEV)
            rdma = pltpu.make_async_remote_copy(
                src_ref=x_ref.at[pl.ds(tgt * BLK, BLK), :],
                dst_ref=xg_ref.at[my],
                send_sem=send_sems.at[s - 1],
                recv_sem=recv_sems.at[s - 1],
                device_id=(tgt,),
                device_id_type=pl.DeviceIdType.MESH,
            )
            rdma.start()
            rdmas.append(rdma)

        xg_ref[my] = x_ref[pl.ds(my * BLK, BLK), :]

        w_copy.wait()

        y = jnp.dot(xg_ref[my], w_ref[pl.ds(my * BLK, BLK), :],
                    preferred_element_type=jnp.float32)
        for o in _OFFSET_ORDER:
            src = lax.rem(my - o + N_DEV, N_DEV)
            recv = pltpu.make_async_remote_copy(
                src_ref=x_ref.at[pl.ds(0, BLK), :],
                dst_ref=xg_ref.at[src],
                send_sem=send_sems.at[0],
                recv_sem=recv_sems.at[o - 1],
                device_id=(my,),
                device_id_type=pl.DeviceIdType.MESH,
            )
            recv.wait_recv()
            y = y + jnp.dot(xg_ref[src], w_ref[pl.ds(src * BLK, BLK), :],
                            preferred_element_type=jnp.float32)

        out_ref[:, :] = jnp.maximum(y, 0.0)

        for rdma in rdmas:
            rdma.wait_send()

    return pl.pallas_call(
        body,
        out_shape=jax.ShapeDtypeStruct((BLK, n), jnp.float32),
        in_specs=[
            pl.BlockSpec(memory_space=pltpu.VMEM),
            pl.BlockSpec(memory_space=pl.ANY),
        ],
        out_specs=pl.BlockSpec(memory_space=pltpu.VMEM),
        scratch_shapes=[
            pltpu.VMEM((N_DEV * BLK, N_DEV * BLK), jnp.float32),
            pltpu.VMEM((N_DEV, BLK, BLK), jnp.float32),
            pltpu.SemaphoreType.DMA,
            pltpu.SemaphoreType.DMA((N_DEV - 1,)),
            pltpu.SemaphoreType.DMA((N_DEV - 1,)),
        ],
        compiler_params=pltpu.CompilerParams(collective_id=0),
    )(x, w_mat)
